# Initial kernel scaffold; baseline (speedup 1.0000x reference)
import functools

import jax
import jax.numpy as jnp
from jax import lax
from jax.experimental import pallas as pl
from jax.experimental.pallas import tpu as pltpu
from jax.experimental.pallas import tpu_sc as plsc

B, C, N, K = 2, 128, 10000, 32
NC, NS, L = 2, 16, 16
SEG = N // NS
TBLK = 2000


def _sc_gather_mean(xt, edges):
    @functools.partial(
        pl.kernel,
        mesh=plsc.VectorSubcoreMesh(core_axis_name="c", subcore_axis_name="s"),
        out_type=jax.ShapeDtypeStruct((B, N, C), jnp.float32),
        scratch_types=[
            pltpu.VMEM_SHARED((N, C), jnp.float32),
            pltpu.VMEM((SEG, K), jnp.int32),
            pltpu.VMEM((SEG, C), jnp.float32),
            pltpu.VMEM((K, C), jnp.float32),
            pltpu.VMEM((K, C), jnp.float32),
            pltpu.SemaphoreType.DMA,
            pltpu.SemaphoreType.DMA,
        ],
    )
    def k(xt_hbm, edges_hbm, out_hbm, x_sh, idx_v, agg_v, r0, r1, sem0, sem1):
        c = lax.axis_index("c")
        s = lax.axis_index("s")
        base = s * SEG

        pltpu.sync_copy(xt_hbm.at[c, pl.ds(base, SEG)], x_sh.at[pl.ds(base, SEG)])
        pltpu.sync_copy(edges_hbm.at[c, pl.ds(base, SEG)], idx_v)
        plsc.subcore_barrier()

        def fire(n, rbuf, sem):
            pltpu.make_async_copy(x_sh.at[idx_v.at[n]], rbuf, sem).start()

        def drain(n, rbuf, sem):
            pltpu.make_async_copy(x_sh.at[idx_v.at[n]], rbuf, sem).wait()

        def reduce_into(n, rbuf):
            for c8 in range(C // L):
                sl = pl.ds(c8 * L, L)
                acc = rbuf[0, sl]
                for j in range(1, K):
                    acc = acc + rbuf[j, sl]
                agg_v[n, sl] = acc * (1.0 / K)

        fire(0, r0, sem0)

        def body(i, carry):
            n = 2 * i
            fire(n + 1, r1, sem1)
            drain(n, r0, sem0)
            reduce_into(n, r0)
            fire(n + 2, r0, sem0)
            drain(n + 1, r1, sem1)
            reduce_into(n + 1, r1)
            return carry

        lax.fori_loop(0, (SEG - 1) // 2, body, 0)
        drain(SEG - 1, r0, sem0)
        reduce_into(SEG - 1, r0)

        pltpu.sync_copy(agg_v, out_hbm.at[c, pl.ds(base, SEG)])

    return k(xt, edges)


def _tc_conv_bn_act(agg, W, gamma2, beta2):
    nblk = N // TBLK

    def body(agg_ref, w_ref, g_ref, b_ref, out_ref, s_acc, m_acc, wp_ref, bp_ref):
        p = pl.program_id(0)
        b = pl.program_id(1)
        j = pl.program_id(2)

        @pl.when(p == 0)
        def _phase_stats():
            blk = agg_ref[0]
            contrib = lax.dot_general(blk, blk, (((0,), (0,)), ((), ())),
                                      preferred_element_type=jnp.float32)
            ones = jnp.ones((TBLK, 1), jnp.float32)
            mcon = lax.dot_general(blk, ones, (((0,), (0,)), ((), ())),
                                   preferred_element_type=jnp.float32)
            first = jnp.logical_and(b == 0, j == 0)

            @pl.when(first)
            def _init():
                s_acc[...] = contrib
                m_acc[...] = mcon

            @pl.when(jnp.logical_not(first))
            def _accum():
                s_acc[...] = s_acc[...] + contrib
                m_acc[...] = m_acc[...] + mcon

        @pl.when(p == 1)
        def _phase_emit():
            @pl.when(jnp.logical_and(b == 0, j == 0))
            def _fold_bn():
                cnt = float(B * N)
                w = w_ref[...]
                mean = lax.dot_general(w, m_acc[...], (((1,), (0,)), ((), ())),
                                       preferred_element_type=jnp.float32) / cnt
                ws = lax.dot_general(w, s_acc[...], (((1,), (0,)), ((), ())),
                                     preferred_element_type=jnp.float32)
                ey2 = jnp.sum(ws * w, axis=1, keepdims=True) / cnt
                var = ey2 - mean * mean
                scale = g_ref[...] * lax.rsqrt(var + 1e-5)
                wp_ref[...] = w * scale
                bp_ref[...] = b_ref[...] - mean * scale

            blk = agg_ref[0]
            y = lax.dot_general(wp_ref[...], blk, (((1,), (1,)), ((), ())),
                                preferred_element_type=jnp.float32)
            y = y + bp_ref[...]
            out_ref[0] = jnp.where(y >= 0, y, 0.2 * y)

    return pl.pallas_call(
        body,
        grid=(2, B, nblk),
        in_specs=[
            pl.BlockSpec((1, TBLK, C), lambda p, b, j: (b, j, 0)),
            pl.BlockSpec((C, C), lambda p, b, j: (0, 0)),
            pl.BlockSpec((C, 1), lambda p, b, j: (0, 0)),
            pl.BlockSpec((C, 1), lambda p, b, j: (0, 0)),
        ],
        out_specs=pl.BlockSpec((1, C, TBLK), lambda p, b, j: (b, 0, j)),
        out_shape=jax.ShapeDtypeStruct((B, C, N), jnp.float32),
        scratch_shapes=[
            pltpu.VMEM((C, C), jnp.float32),
            pltpu.VMEM((C, 1), jnp.float32),
            pltpu.VMEM((C, C), jnp.float32),
            pltpu.VMEM((C, 1), jnp.float32),
        ],
    )(agg, W, gamma2, beta2)


def kernel(x, edges, W, gamma, beta):
    xt = jnp.transpose(x, (0, 2, 1))
    agg = _sc_gather_mean(xt, edges)
    return _tc_conv_bn_act(agg, W, gamma.reshape(C, 1), beta.reshape(C, 1))


# speedup vs baseline: 39.8037x; 39.8037x over previous
import functools

import jax
import jax.numpy as jnp
from jax import lax
from jax.experimental import pallas as pl
from jax.experimental.pallas import tpu as pltpu
from jax.experimental.pallas import tpu_sc as plsc

B, C, N, K = 2, 128, 10000, 32
NC, NS, L = 2, 16, 16
SEG = N // NS
CH = 125
NCH = SEG // CH
TBLK = 2000


def _sc_gather_mean(xt, edges):
    @functools.partial(
        pl.kernel,
        mesh=plsc.VectorSubcoreMesh(core_axis_name="c", subcore_axis_name="s"),
        out_type=jax.ShapeDtypeStruct((B, N, C), jnp.float32),
        compiler_params=pltpu.CompilerParams(use_tc_tiling_on_sc=False),
        scratch_types=[
            pltpu.VMEM_SHARED((N, C), jnp.float32),
            pltpu.VMEM((SEG, K), jnp.int32),
            pltpu.VMEM((CH, C), jnp.float32),
            pltpu.VMEM((K, C), jnp.float32),
            pltpu.VMEM((K, C), jnp.float32),
            pltpu.SemaphoreType.DMA,
            pltpu.SemaphoreType.DMA,
        ],
    )
    def k(xt_hbm, edges_hbm, out_hbm, x_sh, idx_v, agg_v, r0, r1, sem0, sem1):
        c = lax.axis_index("c")
        s = lax.axis_index("s")
        base = s * SEG

        pltpu.sync_copy(xt_hbm.at[c, pl.ds(base, SEG)], x_sh.at[pl.ds(base, SEG)])
        pltpu.sync_copy(edges_hbm.at[c, pl.ds(base, SEG)], idx_v)
        plsc.subcore_barrier()

        def fire(n, rbuf, sem):
            pltpu.make_async_copy(x_sh.at[idx_v.at[n]], rbuf, sem).start()

        def drain(n, rbuf, sem):
            pltpu.make_async_copy(x_sh.at[idx_v.at[n]], rbuf, sem).wait()

        def reduce_into(n, rbuf):
            for c8 in range(C // L):
                sl = pl.ds(c8 * L, L)
                acc = rbuf[0, sl]
                for j in range(1, K):
                    acc = acc + rbuf[j, sl]
                agg_v[n, sl] = acc * (1.0 / K)

        def chunk_body(q, carry):
            start = q * CH
            fire(start, r0, sem0)

            def body(i, cc):
                m = 2 * i
                fire(start + m + 1, r1, sem1)
                drain(start + m, r0, sem0)
                reduce_into(m, r0)
                fire(start + m + 2, r0, sem0)
                drain(start + m + 1, r1, sem1)
                reduce_into(m + 1, r1)
                return cc

            lax.fori_loop(0, (CH - 1) // 2, body, 0)
            drain(start + CH - 1, r0, sem0)
            reduce_into(CH - 1, r0)
            pltpu.sync_copy(agg_v, out_hbm.at[c, pl.ds(base + start, CH)])
            return carry

        lax.fori_loop(0, NCH, chunk_body, 0)

    return k(xt, edges)


def _tc_conv_bn_act(agg, W, gamma2, beta2):
    def body(agg_ref, w_ref, g_ref, b_ref, out_ref, s_acc, m_acc, wp_ref, bp_ref):
        p = pl.program_id(0)
        b = pl.program_id(1)

        @pl.when(p == 0)
        def _phase_stats():
            blk = agg_ref[0]
            contrib = lax.dot_general(blk, blk, (((0,), (0,)), ((), ())),
                                      preferred_element_type=jnp.float32)
            ones = jnp.ones((N, 1), jnp.float32)
            mcon = lax.dot_general(blk, ones, (((0,), (0,)), ((), ())),
                                   preferred_element_type=jnp.float32)

            @pl.when(b == 0)
            def _init():
                s_acc[...] = contrib
                m_acc[...] = mcon

            @pl.when(b != 0)
            def _accum():
                s_acc[...] = s_acc[...] + contrib
                m_acc[...] = m_acc[...] + mcon

        @pl.when(p == 1)
        def _phase_emit():
            @pl.when(b == 0)
            def _fold_bn():
                cnt = float(B * N)
                w = w_ref[...]
                mean = lax.dot_general(w, m_acc[...], (((1,), (0,)), ((), ())),
                                       preferred_element_type=jnp.float32) / cnt
                ws = lax.dot_general(w, s_acc[...], (((1,), (0,)), ((), ())),
                                     preferred_element_type=jnp.float32)
                ey2 = jnp.sum(ws * w, axis=1, keepdims=True) / cnt
                var = ey2 - mean * mean
                scale = g_ref[...] * lax.rsqrt(var + 1e-5)
                wp_ref[...] = w * scale
                bp_ref[...] = b_ref[...] - mean * scale

            for jj in range(N // TBLK):
                blkj = agg_ref[0, pl.ds(jj * TBLK, TBLK), :]
                y = lax.dot_general(wp_ref[...], blkj, (((1,), (1,)), ((), ())),
                                    preferred_element_type=jnp.float32)
                y = y + bp_ref[...]
                out_ref[0, :, pl.ds(jj * TBLK, TBLK)] = jnp.where(y >= 0, y, 0.2 * y)

    return pl.pallas_call(
        body,
        grid=(2, B),
        in_specs=[
            pl.BlockSpec((1, N, C), lambda p, b: (b, 0, 0)),
            pl.BlockSpec((C, C), lambda p, b: (0, 0)),
            pl.BlockSpec((C, 1), lambda p, b: (0, 0)),
            pl.BlockSpec((C, 1), lambda p, b: (0, 0)),
        ],
        out_specs=pl.BlockSpec((1, C, N), lambda p, b: (b, 0, 0)),
        out_shape=jax.ShapeDtypeStruct((B, C, N), jnp.float32),
        scratch_shapes=[
            pltpu.VMEM((C, C), jnp.float32),
            pltpu.VMEM((C, 1), jnp.float32),
            pltpu.VMEM((C, C), jnp.float32),
            pltpu.VMEM((C, 1), jnp.float32),
        ],
    )(agg, W, gamma2, beta2)


def kernel(x, edges, W, gamma, beta):
    xt = jnp.transpose(x, (0, 2, 1))
    agg = _sc_gather_mean(xt, edges)
    return _tc_conv_bn_act(agg, W, gamma.reshape(C, 1), beta.reshape(C, 1))
